# initial kernel scaffold (unmeasured)
import jax
import jax.numpy as jnp
from jax import lax
from jax.experimental import pallas as pl
from jax.experimental.pallas import tpu as pltpu

N_DEV = 4
SQ = 2048
SKV_LOC = 2048
HL = 8
DH = 128
DLOC = HL * DH
NPH = 4
BLK = 64
NG_Q = SQ // (NPH * BLK)
NG_KV = SKV_LOC // (NPH * BLK)
SCALE = 0.08838834764831843


def _a2a_body(k_ref, v_ref, kg_ref, vg_ref, send_sems, recv_sems, copy_sems):
    my = lax.axis_index("i")

    barrier = pltpu.get_barrier_semaphore()
    for di in range(1, N_DEV):
        pl.semaphore_signal(
            barrier, inc=1,
            device_id=((my + di) % N_DEV,),
            device_id_type=pl.DeviceIdType.MESH,
        )
    pl.semaphore_wait(barrier, N_DEV - 1)

    pairs = ((k_ref, kg_ref), (v_ref, vg_ref))

    copies = []
    for t, (src, dst) in enumerate(pairs):
        for p in range(NPH):
            c = pltpu.make_async_copy(
                src.at[:, p, :, pl.ds(my * DLOC, DLOC)],
                dst.at[p, my],
                copy_sems.at[t * NPH + p],
            )
            c.start()
            copies.append(c)

    sends = []
    for di in range(1, N_DEV):
        peer = (my + di) % N_DEV
        for t, (src, dst) in enumerate(pairs):
            for p in range(NPH):
                r = pltpu.make_async_remote_copy(
                    src_ref=src.at[:, p, :, pl.ds(peer * DLOC, DLOC)],
                    dst_ref=dst.at[p, my],
                    send_sem=send_sems.at[((di - 1) * 2 + t) * NPH + p],
                    recv_sem=recv_sems.at[(my * 2 + t) * NPH + p],
                    device_id=(peer,),
                    device_id_type=pl.DeviceIdType.MESH,
                )
                r.start()
                sends.append(r)

    for c in copies:
        c.wait()
    for r in sends:
        r.wait_send()

    for di in range(1, N_DEV):
        src_dev = (my + di) % N_DEV
        for t, (src, dst) in enumerate(pairs):
            for p in range(NPH):
                r = pltpu.make_async_remote_copy(
                    src_ref=src.at[:, p, :, pl.ds(src_dev * DLOC, DLOC)],
                    dst_ref=dst.at[p, src_dev],
                    send_sem=send_sems.at[t * NPH + p],
                    recv_sem=recv_sems.at[(src_dev * 2 + t) * NPH + p],
                    device_id=(src_dev,),
                    device_id_type=pl.DeviceIdType.MESH,
                )
                r.wait_recv()


def _a2a_kv(k4, v4):
    gshape = (NPH, N_DEV, NG_KV, BLK, DLOC)
    return pl.pallas_call(
        _a2a_body,
        out_shape=(
            jax.ShapeDtypeStruct(gshape, jnp.bfloat16),
            jax.ShapeDtypeStruct(gshape, jnp.bfloat16),
        ),
        in_specs=[
            pl.BlockSpec(memory_space=pltpu.ANY),
            pl.BlockSpec(memory_space=pltpu.ANY),
        ],
        out_specs=(
            pl.BlockSpec(memory_space=pltpu.ANY),
            pl.BlockSpec(memory_space=pltpu.ANY),
        ),
        scratch_shapes=[
            pltpu.SemaphoreType.DMA(((N_DEV - 1) * 2 * NPH,)),
            pltpu.SemaphoreType.DMA((N_DEV * 2 * NPH,)),
            pltpu.SemaphoreType.DMA((2 * NPH,)),
        ],
        compiler_params=pltpu.CompilerParams(collective_id=0),
    )(k4, v4)


def _attn_body(x_ref, wq_ref, wo_ref, kg_ref, vg_ref, out_ref,
               kbuf, vbuf, dsem):
    q = jnp.dot(x_ref[:], wq_ref[:], preferred_element_type=jnp.float32)
    q = (q * SCALE).astype(jnp.bfloat16)

    ctx_p = [None] * NPH
    for p in range(NPH):
        ck = pltpu.make_async_copy(kg_ref.at[p], kbuf, dsem.at[0])
        cv = pltpu.make_async_copy(vg_ref.at[p], vbuf, dsem.at[1])
        ck.start()
        cv.start()
        ck.wait()
        cv.wait()
        kp = kbuf[...].reshape(N_DEV * NG_KV * BLK, DLOC)
        vp = vbuf[...].reshape(N_DEV * NG_KV * BLK, DLOC)

        qp = jnp.concatenate(
            [q[g * NPH * BLK + p * BLK: g * NPH * BLK + (p + 1) * BLK, :]
             for g in range(NG_Q)],
            axis=0,
        )

        ctx_h = []
        for h in range(HL):
            qh = qp[:, h * DH:(h + 1) * DH]
            kh = kp[:, h * DH:(h + 1) * DH]
            vh = vp[:, h * DH:(h + 1) * DH]
            s = lax.dot_general(
                qh, kh, (((1,), (1,)), ((), ())),
                preferred_element_type=jnp.float32,
            )
            m = jnp.max(s, axis=1, keepdims=True)
            e = jnp.exp(s - m)
            l = jnp.sum(e, axis=1, keepdims=True)
            w = (e / l).astype(jnp.bfloat16)
            ctx = jnp.dot(w, vh, preferred_element_type=jnp.float32)
            ctx_h.append(ctx.astype(jnp.bfloat16))
        ctx_p[p] = jnp.concatenate(ctx_h, axis=1)

    rows = []
    for g in range(NG_Q):
        for p in range(NPH):
            rows.append(ctx_p[p][g * BLK:(g + 1) * BLK, :])
    ctx = jnp.concatenate(rows, axis=0)

    out = jnp.dot(ctx, wo_ref[:], preferred_element_type=jnp.float32)
    out_ref[:] = out.astype(jnp.bfloat16)


def _attn(x2, wq, wo, kg, vg):
    return pl.pallas_call(
        _attn_body,
        out_shape=jax.ShapeDtypeStruct((SQ, 1024), jnp.bfloat16),
        in_specs=[
            pl.BlockSpec(memory_space=pltpu.VMEM),
            pl.BlockSpec(memory_space=pltpu.VMEM),
            pl.BlockSpec(memory_space=pltpu.VMEM),
            pl.BlockSpec(memory_space=pltpu.ANY),
            pl.BlockSpec(memory_space=pltpu.ANY),
        ],
        out_specs=pl.BlockSpec(memory_space=pltpu.VMEM),
        scratch_shapes=[
            pltpu.VMEM((N_DEV, NG_KV, BLK, DLOC), jnp.bfloat16),
            pltpu.VMEM((N_DEV, NG_KV, BLK, DLOC), jnp.bfloat16),
            pltpu.SemaphoreType.DMA((2,)),
        ],
    )(x2, wq, wo, kg, vg)


def _ar_body(part_ref, out_ref, acc_ref, send_sems, recv_sems, csem):
    my = lax.axis_index("i")

    barrier = pltpu.get_barrier_semaphore()
    for di in range(1, N_DEV):
        pl.semaphore_signal(
            barrier, inc=1,
            device_id=((my + di) % N_DEV,),
            device_id_type=pl.DeviceIdType.MESH,
        )
    pl.semaphore_wait(barrier, N_DEV - 1)

    c = pltpu.make_async_copy(part_ref, acc_ref.at[my], csem)
    c.start()

    sends = []
    for di in range(1, N_DEV):
        peer = (my + di) % N_DEV
        r = pltpu.make_async_remote_copy(
            src_ref=part_ref,
            dst_ref=acc_ref.at[my],
            send_sem=send_sems.at[di - 1],
            recv_sem=recv_sems.at[my],
            device_id=(peer,),
            device_id_type=pl.DeviceIdType.MESH,
        )
        r.start()
        sends.append(r)

    c.wait()
    for r in sends:
        r.wait_send()
    for di in range(1, N_DEV):
        src_dev = (my + di) % N_DEV
        r = pltpu.make_async_remote_copy(
            src_ref=part_ref,
            dst_ref=acc_ref.at[src_dev],
            send_sem=send_sems.at[0],
            recv_sem=recv_sems.at[src_dev],
            device_id=(src_dev,),
            device_id_type=pl.DeviceIdType.MESH,
        )
        r.wait_recv()

    acc = acc_ref[...]
    out_ref[:] = (acc[0].astype(jnp.float32) + acc[1].astype(jnp.float32)
                  + acc[2].astype(jnp.float32) + acc[3].astype(jnp.float32))


def _allreduce(part):
    return pl.pallas_call(
        _ar_body,
        out_shape=jax.ShapeDtypeStruct((SQ, 1024), jnp.float32),
        in_specs=[pl.BlockSpec(memory_space=pltpu.VMEM)],
        out_specs=pl.BlockSpec(memory_space=pltpu.VMEM),
        scratch_shapes=[
            pltpu.VMEM((N_DEV, SQ, 1024), jnp.bfloat16),
            pltpu.SemaphoreType.DMA((N_DEV - 1,)),
            pltpu.SemaphoreType.DMA((N_DEV,)),
            pltpu.SemaphoreType.DMA(()),
        ],
        compiler_params=pltpu.CompilerParams(collective_id=1),
    )(part)


def kernel(x, Wq, K_ext, V_ext, Wo):
    x2 = x[0].astype(jnp.bfloat16)
    wq = Wq.astype(jnp.bfloat16)
    wo = Wo.astype(jnp.bfloat16)
    k4 = K_ext[0].reshape(NG_KV, NPH, BLK, 32 * DH).astype(jnp.bfloat16)
    v4 = V_ext[0].reshape(NG_KV, NPH, BLK, 32 * DH).astype(jnp.bfloat16)

    kg, vg = _a2a_kv(k4, v4)
    part = _attn(x2, wq, wo, kg, vg)
    out = _allreduce(part)
    return out[None]


# baseline (device time: 436077 ns/iter reference)
import jax
import jax.numpy as jnp
from jax import lax
from jax.experimental import pallas as pl
from jax.experimental.pallas import tpu as pltpu

N_DEV = 4
SQ = 2048
SKV_LOC = 2048
HL = 8
DH = 128
DLOC = HL * DH
NPH = 4
BLK = 64
NG_Q = SQ // (NPH * BLK)
NG_KV = SKV_LOC // (NPH * BLK)
SCALE = 0.08838834764831843


def _a2a_body(k_ref, v_ref, kg_ref, vg_ref, send_sems, recv_sems, copy_sems):
    my = lax.axis_index("i")

    barrier = pltpu.get_barrier_semaphore()
    for di in range(1, N_DEV):
        pl.semaphore_signal(
            barrier, inc=1,
            device_id=((my + di) % N_DEV,),
            device_id_type=pl.DeviceIdType.MESH,
        )
    pl.semaphore_wait(barrier, N_DEV - 1)

    pairs = ((k_ref, kg_ref), (v_ref, vg_ref))

    copies = []
    for t, (src, dst) in enumerate(pairs):
        for p in range(NPH):
            c = pltpu.make_async_copy(
                src.at[:, p, :, pl.ds(my * DLOC, DLOC)],
                dst.at[p, my],
                copy_sems.at[t * NPH + p],
            )
            c.start()
            copies.append(c)

    sends = []
    for di in range(1, N_DEV):
        peer = (my + di) % N_DEV
        for t, (src, dst) in enumerate(pairs):
            for p in range(NPH):
                r = pltpu.make_async_remote_copy(
                    src_ref=src.at[:, p, :, pl.ds(peer * DLOC, DLOC)],
                    dst_ref=dst.at[p, my],
                    send_sem=send_sems.at[((di - 1) * 2 + t) * NPH + p],
                    recv_sem=recv_sems.at[(my * 2 + t) * NPH + p],
                    device_id=(peer,),
                    device_id_type=pl.DeviceIdType.MESH,
                )
                r.start()
                sends.append(r)

    for c in copies:
        c.wait()
    for r in sends:
        r.wait_send()

    for di in range(1, N_DEV):
        src_dev = (my + di) % N_DEV
        for t, (src, dst) in enumerate(pairs):
            for p in range(NPH):
                r = pltpu.make_async_remote_copy(
                    src_ref=src.at[:, p, :, pl.ds(src_dev * DLOC, DLOC)],
                    dst_ref=dst.at[p, src_dev],
                    send_sem=send_sems.at[t * NPH + p],
                    recv_sem=recv_sems.at[(src_dev * 2 + t) * NPH + p],
                    device_id=(src_dev,),
                    device_id_type=pl.DeviceIdType.MESH,
                )
                r.wait_recv()


def _a2a_kv(k4, v4):
    gshape = (NPH, N_DEV, NG_KV, BLK, DLOC)
    return pl.pallas_call(
        _a2a_body,
        out_shape=(
            jax.ShapeDtypeStruct(gshape, jnp.bfloat16),
            jax.ShapeDtypeStruct(gshape, jnp.bfloat16),
        ),
        in_specs=[
            pl.BlockSpec(memory_space=pl.ANY),
            pl.BlockSpec(memory_space=pl.ANY),
        ],
        out_specs=(
            pl.BlockSpec(memory_space=pl.ANY),
            pl.BlockSpec(memory_space=pl.ANY),
        ),
        scratch_shapes=[
            pltpu.SemaphoreType.DMA(((N_DEV - 1) * 2 * NPH,)),
            pltpu.SemaphoreType.DMA((N_DEV * 2 * NPH,)),
            pltpu.SemaphoreType.DMA((2 * NPH,)),
        ],
        compiler_params=pltpu.CompilerParams(collective_id=0),
    )(k4, v4)


def _attn_body(x_ref, wq_ref, wo_ref, kg_ref, vg_ref, out_ref,
               kbuf, vbuf, dsem):
    q = jnp.dot(x_ref[:], wq_ref[:], preferred_element_type=jnp.float32)
    q = (q * SCALE).astype(jnp.bfloat16)

    ctx_p = [None] * NPH
    for p in range(NPH):
        ck = pltpu.make_async_copy(kg_ref.at[p], kbuf, dsem.at[0])
        cv = pltpu.make_async_copy(vg_ref.at[p], vbuf, dsem.at[1])
        ck.start()
        cv.start()
        ck.wait()
        cv.wait()
        kp = kbuf[...].reshape(N_DEV * NG_KV * BLK, DLOC)
        vp = vbuf[...].reshape(N_DEV * NG_KV * BLK, DLOC)

        qp = jnp.concatenate(
            [q[g * NPH * BLK + p * BLK: g * NPH * BLK + (p + 1) * BLK, :]
             for g in range(NG_Q)],
            axis=0,
        )

        ctx_h = []
        for h in range(HL):
            qh = qp[:, h * DH:(h + 1) * DH]
            kh = kp[:, h * DH:(h + 1) * DH]
            vh = vp[:, h * DH:(h + 1) * DH]
            s = lax.dot_general(
                qh, kh, (((1,), (1,)), ((), ())),
                preferred_element_type=jnp.float32,
            )
            m = jnp.max(s, axis=1, keepdims=True)
            e = jnp.exp(s - m)
            l = jnp.sum(e, axis=1, keepdims=True)
            w = (e / l).astype(jnp.bfloat16)
            ctx = jnp.dot(w, vh, preferred_element_type=jnp.float32)
            ctx_h.append(ctx.astype(jnp.bfloat16))
        ctx_p[p] = jnp.concatenate(ctx_h, axis=1)

    rows = []
    for g in range(NG_Q):
        for p in range(NPH):
            rows.append(ctx_p[p][g * BLK:(g + 1) * BLK, :])
    ctx = jnp.concatenate(rows, axis=0)

    out = jnp.dot(ctx, wo_ref[:], preferred_element_type=jnp.float32)
    out_ref[:] = out.astype(jnp.bfloat16)


def _attn(x2, wq, wo, kg, vg):
    return pl.pallas_call(
        _attn_body,
        out_shape=jax.ShapeDtypeStruct((SQ, 1024), jnp.bfloat16),
        in_specs=[
            pl.BlockSpec(memory_space=pltpu.VMEM),
            pl.BlockSpec(memory_space=pltpu.VMEM),
            pl.BlockSpec(memory_space=pltpu.VMEM),
            pl.BlockSpec(memory_space=pl.ANY),
            pl.BlockSpec(memory_space=pl.ANY),
        ],
        out_specs=pl.BlockSpec(memory_space=pltpu.VMEM),
        scratch_shapes=[
            pltpu.VMEM((N_DEV, NG_KV, BLK, DLOC), jnp.bfloat16),
            pltpu.VMEM((N_DEV, NG_KV, BLK, DLOC), jnp.bfloat16),
            pltpu.SemaphoreType.DMA((2,)),
        ],
    )(x2, wq, wo, kg, vg)


def _ar_body(part_ref, out_ref, acc_ref, send_sems, recv_sems, csem):
    my = lax.axis_index("i")

    barrier = pltpu.get_barrier_semaphore()
    for di in range(1, N_DEV):
        pl.semaphore_signal(
            barrier, inc=1,
            device_id=((my + di) % N_DEV,),
            device_id_type=pl.DeviceIdType.MESH,
        )
    pl.semaphore_wait(barrier, N_DEV - 1)

    c = pltpu.make_async_copy(part_ref, acc_ref.at[my], csem)
    c.start()

    sends = []
    for di in range(1, N_DEV):
        peer = (my + di) % N_DEV
        r = pltpu.make_async_remote_copy(
            src_ref=part_ref,
            dst_ref=acc_ref.at[my],
            send_sem=send_sems.at[di - 1],
            recv_sem=recv_sems.at[my],
            device_id=(peer,),
            device_id_type=pl.DeviceIdType.MESH,
        )
        r.start()
        sends.append(r)

    c.wait()
    for r in sends:
        r.wait_send()
    for di in range(1, N_DEV):
        src_dev = (my + di) % N_DEV
        r = pltpu.make_async_remote_copy(
            src_ref=part_ref,
            dst_ref=acc_ref.at[src_dev],
            send_sem=send_sems.at[0],
            recv_sem=recv_sems.at[src_dev],
            device_id=(src_dev,),
            device_id_type=pl.DeviceIdType.MESH,
        )
        r.wait_recv()

    acc = acc_ref[...]
    out_ref[:] = (acc[0].astype(jnp.float32) + acc[1].astype(jnp.float32)
                  + acc[2].astype(jnp.float32) + acc[3].astype(jnp.float32))


def _allreduce(part):
    return pl.pallas_call(
        _ar_body,
        out_shape=jax.ShapeDtypeStruct((SQ, 1024), jnp.float32),
        in_specs=[pl.BlockSpec(memory_space=pltpu.VMEM)],
        out_specs=pl.BlockSpec(memory_space=pltpu.VMEM),
        scratch_shapes=[
            pltpu.VMEM((N_DEV, SQ, 1024), jnp.bfloat16),
            pltpu.SemaphoreType.DMA((N_DEV - 1,)),
            pltpu.SemaphoreType.DMA((N_DEV,)),
            pltpu.SemaphoreType.DMA(()),
        ],
        compiler_params=pltpu.CompilerParams(collective_id=1),
    )(part)


def kernel(x, Wq, K_ext, V_ext, Wo):
    x2 = x[0].astype(jnp.bfloat16)
    wq = Wq.astype(jnp.bfloat16)
    wo = Wo.astype(jnp.bfloat16)
    k4 = K_ext[0].reshape(NG_KV, NPH, BLK, 32 * DH).astype(jnp.bfloat16)
    v4 = V_ext[0].reshape(NG_KV, NPH, BLK, 32 * DH).astype(jnp.bfloat16)

    kg, vg = _a2a_kv(k4, v4)
    part = _attn(x2, wq, wo, kg, vg)
    out = _allreduce(part)
    return out[None]


# device time: 335390 ns/iter; 1.3002x vs baseline; 1.3002x over previous
import jax
import jax.numpy as jnp
from jax import lax
from jax.experimental import pallas as pl
from jax.experimental.pallas import tpu as pltpu

N_DEV = 4
SQ = 2048
SKV_LOC = 2048
HL = 8
DH = 128
DLOC = HL * DH
NPH = 4
BLK = 64
NG_Q = SQ // (NPH * BLK)
NG_KV = SKV_LOC // (NPH * BLK)
QROWS = NG_Q * BLK
KROWS = N_DEV * NG_KV * BLK
SCALE = 0.08838834764831843


def _fused_body(x_ref, wq_ref, wo_ref, k_ref, v_ref, out_ref,
                kgbuf, vgbuf, send_sems, recv_sems, copy_sems):
    my = lax.axis_index("i")

    barrier = pltpu.get_barrier_semaphore()
    for di in range(1, N_DEV):
        pl.semaphore_signal(
            barrier, inc=1,
            device_id=((my + di) % N_DEV,),
            device_id_type=pl.DeviceIdType.MESH,
        )
    pl.semaphore_wait(barrier, N_DEV - 1)

    pairs = ((k_ref, kgbuf), (v_ref, vgbuf))

    copies = {}
    sends = []
    recvs = {}
    for p in range(NPH):
        for t, (src, dst) in enumerate(pairs):
            c = pltpu.make_async_copy(
                src.at[:, p, :, pl.ds(my * DLOC, DLOC)],
                dst.at[p, my],
                copy_sems.at[t * NPH + p],
            )
            c.start()
            copies[(t, p)] = c
        for di in range(1, N_DEV):
            peer = (my + di) % N_DEV
            for t, (src, dst) in enumerate(pairs):
                r = pltpu.make_async_remote_copy(
                    src_ref=src.at[:, p, :, pl.ds(peer * DLOC, DLOC)],
                    dst_ref=dst.at[p, my],
                    send_sem=send_sems.at[((di - 1) * 2 + t) * NPH + p],
                    recv_sem=recv_sems.at[(my * 2 + t) * NPH + p],
                    device_id=(peer,),
                    device_id_type=pl.DeviceIdType.MESH,
                )
                r.start()
                sends.append(r)
                src_dev = (my + di) % N_DEV
                rr = pltpu.make_async_remote_copy(
                    src_ref=src.at[:, p, :, pl.ds(src_dev * DLOC, DLOC)],
                    dst_ref=dst.at[p, src_dev],
                    send_sem=send_sems.at[t * NPH + p],
                    recv_sem=recv_sems.at[(src_dev * 2 + t) * NPH + p],
                    device_id=(src_dev,),
                    device_id_type=pl.DeviceIdType.MESH,
                )
                recvs[(di, t, p)] = rr

    q = jnp.dot(x_ref[:], wq_ref[:], preferred_element_type=jnp.float32)
    q = (q * SCALE).astype(jnp.bfloat16)

    ctx_p = [None] * NPH
    for p in range(NPH):
        for t in range(2):
            copies[(t, p)].wait()
            for di in range(1, N_DEV):
                recvs[(di, t, p)].wait_recv()
        kp = kgbuf[p].reshape(KROWS, DLOC)
        vp = vgbuf[p].reshape(KROWS, DLOC)

        qp = jnp.concatenate(
            [q[g * NPH * BLK + p * BLK: g * NPH * BLK + (p + 1) * BLK, :]
             for g in range(NG_Q)],
            axis=0,
        )

        ctx_h = []
        for h in range(HL):
            qh = qp[:, h * DH:(h + 1) * DH]
            kh = kp[:, h * DH:(h + 1) * DH]
            vh = vp[:, h * DH:(h + 1) * DH]
            s = lax.dot_general(
                qh, kh, (((1,), (1,)), ((), ())),
                preferred_element_type=jnp.float32,
            )
            m = jnp.max(s, axis=1, keepdims=True)
            e = jnp.exp(s - m)
            l = jnp.sum(e, axis=1, keepdims=True)
            w = (e / l).astype(jnp.bfloat16)
            ctx = jnp.dot(w, vh, preferred_element_type=jnp.float32)
            ctx_h.append(ctx.astype(jnp.bfloat16))
        ctx_p[p] = jnp.concatenate(ctx_h, axis=1)

    rows = []
    for g in range(NG_Q):
        for p in range(NPH):
            rows.append(ctx_p[p][g * BLK:(g + 1) * BLK, :])
    ctx = jnp.concatenate(rows, axis=0)

    out = jnp.dot(ctx, wo_ref[:], preferred_element_type=jnp.float32)
    out_ref[:] = out.astype(jnp.bfloat16)

    for r in sends:
        r.wait_send()


def _fused_attn(x2, wq, wo, k4, v4):
    gshape = (NPH, N_DEV, NG_KV, BLK, DLOC)
    return pl.pallas_call(
        _fused_body,
        out_shape=jax.ShapeDtypeStruct((SQ, 1024), jnp.bfloat16),
        in_specs=[
            pl.BlockSpec(memory_space=pltpu.VMEM),
            pl.BlockSpec(memory_space=pltpu.VMEM),
            pl.BlockSpec(memory_space=pltpu.VMEM),
            pl.BlockSpec(memory_space=pl.ANY),
            pl.BlockSpec(memory_space=pl.ANY),
        ],
        out_specs=pl.BlockSpec(memory_space=pltpu.VMEM),
        scratch_shapes=[
            pltpu.VMEM(gshape, jnp.bfloat16),
            pltpu.VMEM(gshape, jnp.bfloat16),
            pltpu.SemaphoreType.DMA(((N_DEV - 1) * 2 * NPH,)),
            pltpu.SemaphoreType.DMA((N_DEV * 2 * NPH,)),
            pltpu.SemaphoreType.DMA((2 * NPH,)),
        ],
        compiler_params=pltpu.CompilerParams(
            collective_id=0, vmem_limit_bytes=64 * 1024 * 1024),
    )(x2, wq, wo, k4, v4)


QR = SQ // N_DEV


def _ar_body(part_ref, out_ref, rsbuf, rs_send, rs_recv,
             ag_send, ag_recv, csem):
    my = lax.axis_index("i")

    barrier = pltpu.get_barrier_semaphore()
    for di in range(1, N_DEV):
        pl.semaphore_signal(
            barrier, inc=1,
            device_id=((my + di) % N_DEV,),
            device_id_type=pl.DeviceIdType.MESH,
        )
    pl.semaphore_wait(barrier, N_DEV - 1)

    c = pltpu.make_async_copy(
        part_ref.at[pl.ds(my * QR, QR)], rsbuf.at[my], csem)
    c.start()
    rs_sends = []
    for di in range(1, N_DEV):
        peer = (my + di) % N_DEV
        r = pltpu.make_async_remote_copy(
            src_ref=part_ref.at[pl.ds(peer * QR, QR)],
            dst_ref=rsbuf.at[my],
            send_sem=rs_send.at[di - 1],
            recv_sem=rs_recv.at[my],
            device_id=(peer,),
            device_id_type=pl.DeviceIdType.MESH,
        )
        r.start()
        rs_sends.append(r)

    c.wait()
    for di in range(1, N_DEV):
        src_dev = (my + di) % N_DEV
        r = pltpu.make_async_remote_copy(
            src_ref=part_ref.at[pl.ds(0, QR)],
            dst_ref=rsbuf.at[src_dev],
            send_sem=rs_send.at[0],
            recv_sem=rs_recv.at[src_dev],
            device_id=(src_dev,),
            device_id_type=pl.DeviceIdType.MESH,
        )
        r.wait_recv()
    for r in rs_sends:
        r.wait_send()

    acc = rsbuf[...]
    red = (acc[0].astype(jnp.float32) + acc[1].astype(jnp.float32)
           + acc[2].astype(jnp.float32) + acc[3].astype(jnp.float32))
    out_ref[pl.ds(my * QR, QR), :] = red.astype(jnp.bfloat16)

    ag_sends = []
    for di in range(1, N_DEV):
        peer = (my + di) % N_DEV
        r = pltpu.make_async_remote_copy(
            src_ref=out_ref.at[pl.ds(my * QR, QR)],
            dst_ref=out_ref.at[pl.ds(my * QR, QR)],
            send_sem=ag_send.at[di - 1],
            recv_sem=ag_recv.at[my],
            device_id=(peer,),
            device_id_type=pl.DeviceIdType.MESH,
        )
        r.start()
        ag_sends.append(r)
    for di in range(1, N_DEV):
        src_dev = (my + di) % N_DEV
        r = pltpu.make_async_remote_copy(
            src_ref=out_ref.at[pl.ds(src_dev * QR, QR)],
            dst_ref=out_ref.at[pl.ds(src_dev * QR, QR)],
            send_sem=ag_send.at[0],
            recv_sem=ag_recv.at[src_dev],
            device_id=(src_dev,),
            device_id_type=pl.DeviceIdType.MESH,
        )
        r.wait_recv()
    for r in ag_sends:
        r.wait_send()


def _allreduce(part):
    return pl.pallas_call(
        _ar_body,
        out_shape=jax.ShapeDtypeStruct((SQ, 1024), jnp.bfloat16),
        in_specs=[pl.BlockSpec(memory_space=pltpu.VMEM)],
        out_specs=pl.BlockSpec(memory_space=pltpu.VMEM),
        scratch_shapes=[
            pltpu.VMEM((N_DEV, QR, 1024), jnp.bfloat16),
            pltpu.SemaphoreType.DMA((N_DEV - 1,)),
            pltpu.SemaphoreType.DMA((N_DEV,)),
            pltpu.SemaphoreType.DMA((N_DEV - 1,)),
            pltpu.SemaphoreType.DMA((N_DEV,)),
            pltpu.SemaphoreType.DMA(()),
        ],
        compiler_params=pltpu.CompilerParams(collective_id=1),
    )(part)


def kernel(x, Wq, K_ext, V_ext, Wo):
    x2 = x[0].astype(jnp.bfloat16)
    wq = Wq.astype(jnp.bfloat16)
    wo = Wo.astype(jnp.bfloat16)
    k4 = K_ext[0].reshape(NG_KV, NPH, BLK, 32 * DH).astype(jnp.bfloat16)
    v4 = V_ext[0].reshape(NG_KV, NPH, BLK, 32 * DH).astype(jnp.bfloat16)

    part = _fused_attn(x2, wq, wo, k4, v4)
    out = _allreduce(part)
    return out[None].astype(jnp.float32)
